# baseline (speedup 1.0000x reference)
import functools

import jax
import jax.numpy as jnp
import numpy as np
from jax.experimental import pallas as pl
from jax.experimental.pallas import tpu as pltpu


_ROT1 = (13, 15, 26, 6)
_ROT2 = (17, 29, 16, 24)


def _rotl(x, d):
    return (x << jnp.uint32(d)) | (x >> jnp.uint32(32 - d))


def _rounds(x0, x1, rots):
    for r in rots:
        x0 = x0 + x1
        x1 = _rotl(x1, r)
        x1 = x0 ^ x1
    return x0, x1


def _threefry_bits(x1, k1, k2):
    assert k1 == 0
    ks0 = jnp.uint32(k1)
    ks1 = jnp.uint32(k2)
    ks2 = jnp.uint32(np.uint32(k1) ^ np.uint32(k2) ^ np.uint32(0x1BD11BDA))
    x0 = x1
    x1 = _rotl(x1, _ROT1[0])
    x1 = x0 ^ x1
    x0, x1 = _rounds(x0, x1, _ROT1[1:])
    x0, x1 = x0 + ks1, x1 + (ks2 + jnp.uint32(1))
    x0, x1 = _rounds(x0, x1, _ROT2)
    x0, x1 = x0 + ks2, x1 + (ks0 + jnp.uint32(2))
    x0, x1 = _rounds(x0, x1, _ROT1)
    x0, x1 = x0 + ks0, x1 + (ks1 + jnp.uint32(3))
    x0, x1 = _rounds(x0, x1, _ROT2)
    x0, x1 = x0 + ks1, x1 + (ks2 + jnp.uint32(4))
    x0, x1 = _rounds(x0, x1, _ROT1)
    x0, x1 = x0 + ks2, x1 + (ks0 + jnp.uint32(5))
    return x0 ^ x1


_NEG_LN2 = np.float32(-np.log(2.0))


def _gumbel_from_bits(bits):
    fbits = (bits >> jnp.uint32(9)) | jnp.uint32(0x3F800000)
    floats = jax.lax.bitcast_convert_type(fbits, jnp.float32) - jnp.float32(1.0)
    tiny = np.float32(np.finfo(np.float32).tiny)
    u = floats + tiny
    return -jnp.log(-jnp.log(u))


def _sample_kernel(x_ref, val_ref, idx_ref, acc_val, acc_blk, *,
                   ncols, block_cols, half_blocks, total_blocks):
    h = pl.program_id(0)
    j = pl.program_id(1)
    nblocks = pl.num_programs(1)
    rows, cols = x_ref.shape
    blk = h * half_blocks + j

    @pl.when(j == 0)
    def _():
        acc_val[...] = jnp.full((rows, cols), -jnp.inf, jnp.float32)
        acc_blk[...] = jnp.zeros((rows, cols), jnp.int32)

    tail = ncols % block_cols
    if tail:
        @pl.when(blk == total_blocks - 1)
        def _():
            x_ref[:, tail:] = jnp.full((rows, cols - tail), -jnp.inf,
                                       jnp.float32)

    @pl.when(blk > total_blocks - 1)
    def _():
        x_ref[...] = jnp.full((rows, cols), -jnp.inf, jnp.float32)

    col = jax.lax.broadcasted_iota(jnp.uint32, (rows, cols), 1) + (
        blk * block_cols + 42).astype(jnp.uint32)
    row = jax.lax.broadcasted_iota(jnp.uint32, (rows, cols), 0)
    x1 = row * jnp.uint32(ncols) + col

    gumbel = _gumbel_from_bits(_threefry_bits(x1, 0, 42))
    vals = gumbel + x_ref[...]

    upd = vals > acc_val[...]
    acc_blk[...] = jnp.where(upd, blk, acc_blk[...])
    acc_val[...] = jnp.where(upd, vals, acc_val[...])

    @pl.when(j == nblocks - 1)
    def _():
        av = acc_val[...]
        m = jnp.max(av, axis=1, keepdims=True)
        pos = jax.lax.broadcasted_iota(jnp.int32, (rows, cols), 1)
        cand = jnp.where(
            av == m,
            acc_blk[...] * block_cols + pos,
            jnp.int32(np.iinfo(np.int32).max),
        )
        val_ref[...] = m
        idx_ref[...] = jnp.min(cand, axis=1, keepdims=True)


@jax.jit
def kernel(log_p):
    rows, ncols = log_p.shape
    block_cols = 8192
    total_blocks = pl.cdiv(ncols, block_cols)
    half_blocks = pl.cdiv(total_blocks, 2)
    vals, idx = pl.pallas_call(
        functools.partial(_sample_kernel, ncols=ncols, block_cols=block_cols,
                          half_blocks=half_blocks, total_blocks=total_blocks),
        grid=(2, half_blocks),
        in_specs=[pl.BlockSpec(
            (rows, block_cols),
            lambda h, j: (0, jnp.minimum(h * half_blocks + j,
                                         total_blocks - 1)))],
        out_specs=[
            pl.BlockSpec((rows, 1), lambda h, j: (h, 0)),
            pl.BlockSpec((rows, 1), lambda h, j: (h, 0)),
        ],
        out_shape=[
            jax.ShapeDtypeStruct((2 * rows, 1), jnp.float32),
            jax.ShapeDtypeStruct((2 * rows, 1), jnp.int32),
        ],
        scratch_shapes=[
            pltpu.VMEM((rows, block_cols), jnp.float32),
            pltpu.VMEM((rows, block_cols), jnp.int32),
        ],
        compiler_params=pltpu.CompilerParams(
            dimension_semantics=("parallel", "arbitrary"),
        ),
    )(log_p)
    v = vals.reshape(2, rows)
    ix = idx.reshape(2, rows)
    out = jnp.where(v[1] > v[0], ix[1], ix[0])
    return out.astype(jnp.int64)


# speedup vs baseline: 1.0197x; 1.0197x over previous
import functools

import jax
import jax.numpy as jnp
import numpy as np
from jax.experimental import pallas as pl
from jax.experimental.pallas import tpu as pltpu


_ROT1 = (13, 15, 26, 6)
_ROT2 = (17, 29, 16, 24)


def _rotl(x, d):
    return (x << jnp.uint32(d)) | (x >> jnp.uint32(32 - d))


def _rounds(x0, x1, rots):
    for r in rots:
        x0 = x0 + x1
        x1 = _rotl(x1, r)
        x1 = x0 ^ x1
    return x0, x1


def _threefry_bits(x1, k1, k2):
    assert k1 == 0
    ks0 = jnp.uint32(k1)
    ks1 = jnp.uint32(k2)
    ks2 = jnp.uint32(np.uint32(k1) ^ np.uint32(k2) ^ np.uint32(0x1BD11BDA))
    x0 = x1
    x1 = _rotl(x1, _ROT1[0])
    x1 = x0 ^ x1
    x0, x1 = _rounds(x0, x1, _ROT1[1:])
    x0, x1 = x0 + ks1, x1 + (ks2 + jnp.uint32(1))
    x0, x1 = _rounds(x0, x1, _ROT2)
    x0, x1 = x0 + ks2, x1 + (ks0 + jnp.uint32(2))
    x0, x1 = _rounds(x0, x1, _ROT1)
    x0, x1 = x0 + ks0, x1 + (ks1 + jnp.uint32(3))
    x0, x1 = _rounds(x0, x1, _ROT2)
    x0, x1 = x0 + ks1, x1 + (ks2 + jnp.uint32(4))
    x0, x1 = _rounds(x0, x1, _ROT1)
    x0, x1 = x0 + ks2, x1 + (ks0 + jnp.uint32(5))
    return x0 ^ x1


_NEG_LN2 = np.float32(-np.log(2.0))


def _gumbel_from_bits(bits):
    fbits = (bits >> jnp.uint32(9)) | jnp.uint32(0x3F800000)
    floats = jax.lax.bitcast_convert_type(fbits, jnp.float32) - jnp.float32(1.0)
    tiny = np.float32(np.finfo(np.float32).tiny)
    u = floats + tiny
    return -jnp.log(-jnp.log(u))


def _sample_kernel(x_ref, idx_ref, acc_val, acc_blk, *, ncols, block_cols):
    j = pl.program_id(0)
    nblocks = pl.num_programs(0)
    rows, cols = x_ref.shape

    @pl.when(j == 0)
    def _():
        acc_val[...] = jnp.full((rows, cols), -jnp.inf, jnp.float32)
        acc_blk[...] = jnp.zeros((rows, cols), jnp.int32)

    tail = ncols % block_cols
    if tail:
        @pl.when(j == nblocks - 1)
        def _():
            x_ref[:, tail:] = jnp.full((rows, cols - tail), -jnp.inf, jnp.float32)

    col = jax.lax.broadcasted_iota(jnp.uint32, (rows, cols), 1) + (
        j * block_cols + 42).astype(jnp.uint32)
    row = jax.lax.broadcasted_iota(jnp.uint32, (rows, cols), 0)
    x1 = row * jnp.uint32(ncols) + col

    gumbel = _gumbel_from_bits(_threefry_bits(x1, 0, 42))
    vals = gumbel + x_ref[...]

    upd = vals > acc_val[...]
    acc_blk[...] = jnp.where(upd, j, acc_blk[...])
    acc_val[...] = jnp.where(upd, vals, acc_val[...])

    @pl.when(j == nblocks - 1)
    def _():
        av = acc_val[...]
        m = jnp.max(av, axis=1, keepdims=True)
        pos = jax.lax.broadcasted_iota(jnp.int32, (rows, cols), 1)
        cand = jnp.where(
            av == m,
            acc_blk[...] * block_cols + pos,
            jnp.int32(np.iinfo(np.int32).max),
        )
        idx_ref[...] = jnp.min(cand, axis=1, keepdims=True)


@jax.jit
def kernel(log_p):
    rows, ncols = log_p.shape
    block_cols = 12288
    grid = pl.cdiv(ncols, block_cols)
    idx = pl.pallas_call(
        functools.partial(_sample_kernel, ncols=ncols, block_cols=block_cols),
        grid=(grid,),
        in_specs=[pl.BlockSpec((rows, block_cols), lambda j: (0, j))],
        out_specs=pl.BlockSpec((rows, 1), lambda j: (0, 0)),
        out_shape=jax.ShapeDtypeStruct((rows, 1), jnp.int32),
        scratch_shapes=[
            pltpu.VMEM((rows, block_cols), jnp.float32),
            pltpu.VMEM((rows, block_cols), jnp.int32),
        ],
        compiler_params=pltpu.CompilerParams(
            dimension_semantics=("arbitrary",),
        ),
    )(log_p)
    return idx[:, 0].astype(jnp.int64)


# speedup vs baseline: 1.0279x; 1.0080x over previous
import functools

import jax
import jax.numpy as jnp
import numpy as np
from jax.experimental import pallas as pl
from jax.experimental.pallas import tpu as pltpu


_ROT1 = (13, 15, 26, 6)
_ROT2 = (17, 29, 16, 24)


def _rotl(x, d):
    return (x << jnp.uint32(d)) | (x >> jnp.uint32(32 - d))


def _rounds(x0, x1, rots):
    for r in rots:
        x0 = x0 + x1
        x1 = _rotl(x1, r)
        x1 = x0 ^ x1
    return x0, x1


def _threefry_bits(x1, k1, k2):
    assert k1 == 0
    ks0 = jnp.uint32(k1)
    ks1 = jnp.uint32(k2)
    ks2 = jnp.uint32(np.uint32(k1) ^ np.uint32(k2) ^ np.uint32(0x1BD11BDA))
    x0 = x1
    x1 = _rotl(x1, _ROT1[0])
    x1 = x0 ^ x1
    x0, x1 = _rounds(x0, x1, _ROT1[1:])
    x0, x1 = x0 + ks1, x1 + (ks2 + jnp.uint32(1))
    x0, x1 = _rounds(x0, x1, _ROT2)
    x0, x1 = x0 + ks2, x1 + (ks0 + jnp.uint32(2))
    x0, x1 = _rounds(x0, x1, _ROT1)
    x0, x1 = x0 + ks0, x1 + (ks1 + jnp.uint32(3))
    x0, x1 = _rounds(x0, x1, _ROT2)
    x0, x1 = x0 + ks1, x1 + (ks2 + jnp.uint32(4))
    x0, x1 = _rounds(x0, x1, _ROT1)
    x0, x1 = x0 + ks2, x1 + (ks0 + jnp.uint32(5))
    return x0 ^ x1


_NEG_LN2 = np.float32(-np.log(2.0))


def _gumbel_from_bits(bits):
    fbits = (bits >> jnp.uint32(9)) | jnp.uint32(0x3F800000)
    u = jax.lax.bitcast_convert_type(fbits, jnp.float32) - jnp.float32(1.0)
    return -jnp.log(-jnp.log(u))


def _sample_kernel(x_ref, idx_ref, acc_val, acc_blk, *, ncols, block_cols):
    j = pl.program_id(0)
    nblocks = pl.num_programs(0)
    rows, cols = x_ref.shape

    @pl.when(j == 0)
    def _():
        acc_val[...] = jnp.full((rows, cols), -jnp.inf, jnp.float32)
        acc_blk[...] = jnp.zeros((rows, cols), jnp.int32)

    tail = ncols % block_cols
    if tail:
        @pl.when(j == nblocks - 1)
        def _():
            x_ref[:, tail:] = jnp.full((rows, cols - tail), -jnp.inf, jnp.float32)

    col = jax.lax.broadcasted_iota(jnp.uint32, (rows, cols), 1) + (
        j * block_cols + 42).astype(jnp.uint32)
    row = jax.lax.broadcasted_iota(jnp.uint32, (rows, cols), 0)
    x1 = row * jnp.uint32(ncols) + col

    gumbel = _gumbel_from_bits(_threefry_bits(x1, 0, 42))
    vals = gumbel + x_ref[...]

    upd = vals > acc_val[...]
    acc_blk[...] = jnp.where(upd, j, acc_blk[...])
    acc_val[...] = jnp.where(upd, vals, acc_val[...])

    @pl.when(j == nblocks - 1)
    def _():
        av = acc_val[...]
        m = jnp.max(av, axis=1, keepdims=True)
        pos = jax.lax.broadcasted_iota(jnp.int32, (rows, cols), 1)
        cand = jnp.where(
            av == m,
            acc_blk[...] * block_cols + pos,
            jnp.int32(np.iinfo(np.int32).max),
        )
        idx_ref[...] = jnp.min(cand, axis=1, keepdims=True)


@jax.jit
def kernel(log_p):
    rows, ncols = log_p.shape
    block_cols = 12288
    grid = pl.cdiv(ncols, block_cols)
    idx = pl.pallas_call(
        functools.partial(_sample_kernel, ncols=ncols, block_cols=block_cols),
        grid=(grid,),
        in_specs=[pl.BlockSpec((rows, block_cols), lambda j: (0, j))],
        out_specs=pl.BlockSpec((rows, 1), lambda j: (0, 0)),
        out_shape=jax.ShapeDtypeStruct((rows, 1), jnp.int32),
        scratch_shapes=[
            pltpu.VMEM((rows, block_cols), jnp.float32),
            pltpu.VMEM((rows, block_cols), jnp.int32),
        ],
        compiler_params=pltpu.CompilerParams(
            dimension_semantics=("arbitrary",),
        ),
    )(log_p)
    return idx[:, 0].astype(jnp.int64)
